# baseline (speedup 1.0000x reference)
import functools

import jax
import jax.numpy as jnp
import numpy as np
from jax.experimental import pallas as pl
from jax.experimental.pallas import tpu as pltpu

B, N, D = 16, 576, 768
NTE, NCE, TOPK = 4, 4, 2
E = NTE + NCE
HT = N * 4
HC = D * 4
EPS = 1e-6
S = B * TOPK
J = 2
HT_T = HT // J
HC_T = HC // J


def _gelu(x):
    return 0.5 * x * (1.0 + jnp.tanh(np.sqrt(2.0 / np.pi) * (x + 0.044715 * x ** 3)))


def _router_body(x_ref, rw_ref, idx_ref, wts_ref, aux_ref, ri_ref):
    b = pl.program_id(0)
    ri_ref[b, :] = jnp.mean(x_ref[0], axis=0)

    @pl.when(b == B - 1)
    def _():
        ri = ri_ref[...]
        logits = jax.lax.dot_general(
            ri, rw_ref[...], (((1,), (1,)), ((), ())),
            preferred_element_type=jnp.float32)
        m = jnp.max(logits, axis=-1, keepdims=True)
        p = jnp.exp(logits - m)
        p = p / jnp.sum(p, axis=-1, keepdims=True)
        colid = jax.lax.broadcasted_iota(jnp.int32, (B, E), 1)
        v1 = jnp.max(p, axis=-1, keepdims=True)
        i1 = jnp.min(jnp.where(p == v1, colid, E), axis=-1, keepdims=True)
        pm = jnp.where(colid == i1, -jnp.inf, p)
        v2 = jnp.max(pm, axis=-1, keepdims=True)
        i2 = jnp.min(jnp.where(pm == v2, colid, E), axis=-1, keepdims=True)
        s = v1 + v2
        idx_ref[...] = jnp.concatenate([i1, i2], axis=1)
        wts_ref[...] = jnp.concatenate([v1 / s, v2 / s], axis=1)
        ep = jnp.mean(p, axis=0, keepdims=True)
        ef = jnp.mean(jnp.where(colid == i1, 1.0, 0.0), axis=0, keepdims=True)
        aux_ref[...] = E * jnp.sum(ep * ef, axis=1, keepdims=True)


def _router_call(x, router_w):
    return pl.pallas_call(
        _router_body,
        grid=(B,),
        in_specs=[
            pl.BlockSpec((1, N, D), lambda b: (b, 0, 0)),
            pl.BlockSpec((E, D), lambda b: (0, 0)),
        ],
        out_specs=[
            pl.BlockSpec((B, TOPK), lambda b: (0, 0)),
            pl.BlockSpec((B, TOPK), lambda b: (0, 0)),
            pl.BlockSpec((1, 1), lambda b: (0, 0)),
        ],
        out_shape=[
            jax.ShapeDtypeStruct((B, TOPK), jnp.int32),
            jax.ShapeDtypeStruct((B, TOPK), jnp.float32),
            jax.ShapeDtypeStruct((1, 1), jnp.float32),
        ],
        scratch_shapes=[pltpu.VMEM((B, D), jnp.float32)],
        compiler_params=pltpu.CompilerParams(
            dimension_semantics=("arbitrary",)),
    )(x, router_w)


def _dispatch_body(sb_ref, sw_ref, se_ref, th_ref, ch_ref,
                   x_ref, tw1_ref, tb1_ref, tw2_ref, tb2_ref,
                   cw1_ref, cb1_ref, cw2_ref, cb2_ref,
                   out_ref, xnt_ref, xnc_ref):
    s, j = pl.program_id(0), pl.program_id(1)
    e = se_ref[s]
    w = sw_ref[s]

    @pl.when(e < NTE)
    def _():
        @pl.when(j == 0)
        def _():
            xx = x_ref[0]
            mt = jnp.mean(xx, axis=0, keepdims=True)
            vt = jnp.mean((xx - mt) ** 2, axis=0, keepdims=True)
            xnt_ref[...] = ((xx - mt) * jax.lax.rsqrt(vt + EPS)).astype(jnp.bfloat16)

        h = jax.lax.dot_general(
            tw1_ref[0].astype(jnp.bfloat16), xnt_ref[...], (((1,), (0,)), ((), ())),
            preferred_element_type=jnp.float32)
        g = _gelu(h + tb1_ref[0, 0, :][:, None]).astype(jnp.bfloat16)
        y = jax.lax.dot_general(
            tw2_ref[0].astype(jnp.bfloat16), g, (((1,), (0,)), ((), ())),
            preferred_element_type=jnp.float32)
        contrib = jnp.where(j == 0, w * (y + tb2_ref[0, 0, :][:, None]), w * y)

        @pl.when(j == 0)
        def _():
            out_ref[...] = contrib[None]

        @pl.when(j != 0)
        def _():
            out_ref[...] += contrib[None]

    @pl.when(e >= NTE)
    def _():
        @pl.when(j == 0)
        def _():
            xx = x_ref[0]
            mc = jnp.mean(xx, axis=1, keepdims=True)
            vc = jnp.mean((xx - mc) ** 2, axis=1, keepdims=True)
            xnc_ref[...] = ((xx - mc) * jax.lax.rsqrt(vc + EPS)).astype(jnp.bfloat16)

        h = jax.lax.dot_general(
            xnc_ref[...], cw1_ref[0].astype(jnp.bfloat16), (((1,), (1,)), ((), ())),
            preferred_element_type=jnp.float32)
        g = _gelu(h + cb1_ref[0, 0, :][None, :]).astype(jnp.bfloat16)
        y = jax.lax.dot_general(
            g, cw2_ref[0].astype(jnp.bfloat16), (((1,), (1,)), ((), ())),
            preferred_element_type=jnp.float32)
        contrib = jnp.where(j == 0, w * (y + cb2_ref[0, 0, :][None, :]), w * y)

        @pl.when(j == 0)
        def _():
            out_ref[...] = contrib[None]

        @pl.when(j != 0)
        def _():
            out_ref[...] += contrib[None]


def _hold_prev(is_mine, val):
    pos = jnp.arange(val.shape[0], dtype=jnp.int32)
    ff = jax.lax.cummax(jnp.where(is_mine, pos, -1))
    return jnp.where(ff >= 0, jnp.take(jnp.where(is_mine, val, 0),
                                       jnp.maximum(ff, 0)), 0).astype(jnp.int32)


def _dispatch_call(x, sb, sw, se, tw1, tb1, tw2, tb2, cw1, cb1, cw2, cb2):
    th = _hold_prev(se < NTE, se)
    ch = _hold_prev(se >= NTE, se - NTE)
    grid_spec = pltpu.PrefetchScalarGridSpec(
        num_scalar_prefetch=5,
        grid=(S, J),
        in_specs=[
            pl.BlockSpec((1, N, D), lambda s, j, sb, sw, se, th, ch: (sb[s], 0, 0)),
            pl.BlockSpec((1, HT_T, N), lambda s, j, sb, sw, se, th, ch: (th[s], j, 0)),
            pl.BlockSpec((1, 1, HT_T), lambda s, j, sb, sw, se, th, ch: (th[s] * J + j, 0, 0)),
            pl.BlockSpec((1, N, HT_T), lambda s, j, sb, sw, se, th, ch: (th[s], 0, j)),
            pl.BlockSpec((1, 1, N), lambda s, j, sb, sw, se, th, ch: (th[s], 0, 0)),
            pl.BlockSpec((1, HC_T, D), lambda s, j, sb, sw, se, th, ch: (ch[s], j, 0)),
            pl.BlockSpec((1, 1, HC_T), lambda s, j, sb, sw, se, th, ch: (ch[s] * J + j, 0, 0)),
            pl.BlockSpec((1, D, HC_T), lambda s, j, sb, sw, se, th, ch: (ch[s], 0, j)),
            pl.BlockSpec((1, 1, D), lambda s, j, sb, sw, se, th, ch: (ch[s], 0, 0)),
        ],
        out_specs=pl.BlockSpec((1, N, D), lambda s, j, sb, sw, se, th, ch: (s, 0, 0)),
        scratch_shapes=[
            pltpu.VMEM((N, D), jnp.bfloat16),
            pltpu.VMEM((N, D), jnp.bfloat16),
        ],
    )
    return pl.pallas_call(
        _dispatch_body,
        grid_spec=grid_spec,
        out_shape=jax.ShapeDtypeStruct((S, N, D), jnp.float32),
        compiler_params=pltpu.CompilerParams(
            dimension_semantics=("arbitrary", "arbitrary")),
    )(sb, sw, se, th, ch, x, tw1, tb1, tw2, tb2, cw1, cb1, cw2, cb2)


def _combine_body(s0_ref, s1_ref, a_ref, b_ref, out_ref):
    out_ref[...] = a_ref[...] + b_ref[...]


def _combine_call(slot_out, s0, s1):
    grid_spec = pltpu.PrefetchScalarGridSpec(
        num_scalar_prefetch=2,
        grid=(B,),
        in_specs=[
            pl.BlockSpec((1, N, D), lambda b, s0, s1: (s0[b], 0, 0)),
            pl.BlockSpec((1, N, D), lambda b, s0, s1: (s1[b], 0, 0)),
        ],
        out_specs=pl.BlockSpec((1, N, D), lambda b, s0, s1: (b, 0, 0)),
    )
    return pl.pallas_call(
        _combine_body,
        grid_spec=grid_spec,
        out_shape=jax.ShapeDtypeStruct((B, N, D), jnp.float32),
        compiler_params=pltpu.CompilerParams(
            dimension_semantics=("arbitrary",)),
    )(s0, s1, slot_out, slot_out)


def kernel(x, router_w, tm_fc1_w, tm_fc1_b, tm_fc2_w, tm_fc2_b,
           cm_fc1_w, cm_fc1_b, cm_fc2_w, cm_fc2_b):
    sexp, swts, aux = _router_call(x, router_w)
    se_flat = sexp.reshape(-1)
    order = jnp.argsort(se_flat, stable=True).astype(jnp.int32)
    sb = (order // TOPK).astype(jnp.int32)
    se = jnp.take(se_flat, order)
    sw = jnp.take(swts.reshape(-1), order)
    inv = jnp.argsort(order, stable=True).astype(jnp.int32)
    s0 = inv[0::TOPK]
    s1 = inv[1::TOPK]

    tb1 = tm_fc1_b.reshape(NTE * J, 1, HT_T)
    tb2 = tm_fc2_b.reshape(NTE, 1, N)
    cb1 = cm_fc1_b.reshape(NCE * J, 1, HC_T)
    cb2 = cm_fc2_b.reshape(NCE, 1, D)
    slot_out = _dispatch_call(x, sb, sw, se, tm_fc1_w, tb1, tm_fc2_w, tb2,
                              cm_fc1_w, cb1, cm_fc2_w, cb2)
    out = _combine_call(slot_out, s0, s1)
    return out, aux[0, 0]


# speedup vs baseline: 1.3858x; 1.3858x over previous
import functools

import jax
import jax.numpy as jnp
import numpy as np
from jax.experimental import pallas as pl
from jax.experimental.pallas import tpu as pltpu

B, N, D = 16, 576, 768
NTE, NCE, TOPK = 4, 4, 2
E = NTE + NCE
HT = N * 4
HC = D * 4
EPS = 1e-6
J = 3
HT_T = HT // J
HC_T = HC // J


def _gelu(x):
    return 0.5 * x * (1.0 + jnp.tanh(np.sqrt(2.0 / np.pi) * (x + 0.044715 * x ** 3)))


def _router_body(x_ref, rw_ref, idx_ref, wts_ref, aux_ref, ri_ref):
    b = pl.program_id(0)
    ri_ref[b, :] = jnp.mean(x_ref[0], axis=0)

    @pl.when(b == B - 1)
    def _():
        ri = ri_ref[...]
        logits = jax.lax.dot_general(
            ri, rw_ref[...], (((1,), (1,)), ((), ())),
            preferred_element_type=jnp.float32)
        m = jnp.max(logits, axis=-1, keepdims=True)
        p = jnp.exp(logits - m)
        p = p / jnp.sum(p, axis=-1, keepdims=True)
        colid = jax.lax.broadcasted_iota(jnp.int32, (B, E), 1)
        v1 = jnp.max(p, axis=-1, keepdims=True)
        i1 = jnp.min(jnp.where(p == v1, colid, E), axis=-1, keepdims=True)
        pm = jnp.where(colid == i1, -jnp.inf, p)
        v2 = jnp.max(pm, axis=-1, keepdims=True)
        i2 = jnp.min(jnp.where(pm == v2, colid, E), axis=-1, keepdims=True)
        s = v1 + v2
        idx_ref[...] = jnp.concatenate([i1, i2], axis=1)
        wts_ref[...] = jnp.concatenate([v1 / s, v2 / s], axis=1)
        ep = jnp.mean(p, axis=0, keepdims=True)
        ef = jnp.mean(jnp.where(colid == i1, 1.0, 0.0), axis=0, keepdims=True)
        aux_ref[...] = E * jnp.sum(ep * ef, axis=1, keepdims=True)


def _dispatch_body(sexp_ref, swts_ref, th_ref, ch_ref,
                   x_ref, tw1_ref, tw2_ref, cw1_ref, cw2_ref,
                   out_ref, xnt_ref, xnc_ref):
    b, k = pl.program_id(0), pl.program_id(1)
    e = sexp_ref[b, k]
    w = swts_ref[b, k]

    @pl.when(k == 0)
    def _():
        xx = x_ref[0]
        mt = jnp.mean(xx, axis=0, keepdims=True)
        vt = jnp.mean((xx - mt) ** 2, axis=0, keepdims=True)
        xnt_ref[...] = ((xx - mt) * jax.lax.rsqrt(vt + EPS)).astype(jnp.bfloat16)
        mc = jnp.mean(xx, axis=1, keepdims=True)
        vc = jnp.mean((xx - mc) ** 2, axis=1, keepdims=True)
        xnc_ref[...] = ((xx - mc) * jax.lax.rsqrt(vc + EPS)).astype(jnp.bfloat16)

    @pl.when(e < NTE)
    def _():
        h = jax.lax.dot_general(
            tw1_ref[0], xnt_ref[...], (((1,), (0,)), ((), ())),
            preferred_element_type=jnp.float32)
        g = _gelu(h).astype(jnp.bfloat16)
        y = jax.lax.dot_general(
            tw2_ref[0], g, (((1,), (0,)), ((), ())),
            preferred_element_type=jnp.float32)
        contrib = w * y

        @pl.when(k == 0)
        def _():
            out_ref[...] = contrib[None]

        @pl.when(k != 0)
        def _():
            out_ref[...] += contrib[None]

    @pl.when(e >= NTE)
    def _():
        h = jax.lax.dot_general(
            xnc_ref[...], cw1_ref[0], (((1,), (1,)), ((), ())),
            preferred_element_type=jnp.float32)
        g = _gelu(h).astype(jnp.bfloat16)
        y = jax.lax.dot_general(
            g, cw2_ref[0], (((1,), (1,)), ((), ())),
            preferred_element_type=jnp.float32)
        contrib = w * y

        @pl.when(k == 0)
        def _():
            out_ref[...] = contrib[None]

        @pl.when(k != 0)
        def _():
            out_ref[...] += contrib[None]


def _router_call(x, router_w):
    return pl.pallas_call(
        _router_body,
        grid=(B,),
        in_specs=[
            pl.BlockSpec((1, N, D), lambda b: (b, 0, 0)),
            pl.BlockSpec((E, D), lambda b: (0, 0)),
        ],
        out_specs=[
            pl.BlockSpec((B, TOPK), lambda b: (0, 0)),
            pl.BlockSpec((B, TOPK), lambda b: (0, 0)),
            pl.BlockSpec((1, 1), lambda b: (0, 0)),
        ],
        out_shape=[
            jax.ShapeDtypeStruct((B, TOPK), jnp.int32),
            jax.ShapeDtypeStruct((B, TOPK), jnp.float32),
            jax.ShapeDtypeStruct((1, 1), jnp.float32),
        ],
        scratch_shapes=[pltpu.VMEM((B, D), jnp.float32)],
        compiler_params=pltpu.CompilerParams(
            dimension_semantics=("arbitrary",)),
    )(x, router_w)


def _hold_prev(se_flat, is_mine, val):
    pos = jnp.arange(se_flat.shape[0], dtype=jnp.int32)
    ff = jax.lax.cummax(jnp.where(is_mine, pos, -1))
    return jnp.where(ff >= 0, jnp.take(jnp.where(is_mine, val, 0),
                                       jnp.maximum(ff, 0)), 0).astype(jnp.int32)


def _dispatch_call(x, sexp, swts, tw1, tw2, cw1, cw2):
    se_flat = sexp.reshape(-1)
    th = _hold_prev(se_flat, se_flat < NTE, se_flat).reshape(B, TOPK)
    ch = _hold_prev(se_flat, se_flat >= NTE, se_flat - NTE).reshape(B, TOPK)
    grid_spec = pltpu.PrefetchScalarGridSpec(
        num_scalar_prefetch=4,
        grid=(B, TOPK),
        in_specs=[
            pl.BlockSpec((1, N, D), lambda b, k, se, sw, th, ch: (b, 0, 0)),
            pl.BlockSpec((1, HT, N), lambda b, k, se, sw, th, ch: (th[b, k], 0, 0)),
            pl.BlockSpec((1, N, HT), lambda b, k, se, sw, th, ch: (th[b, k], 0, 0)),
            pl.BlockSpec((1, HC, D), lambda b, k, se, sw, th, ch: (ch[b, k], 0, 0)),
            pl.BlockSpec((1, D, HC), lambda b, k, se, sw, th, ch: (ch[b, k], 0, 0)),
        ],
        out_specs=pl.BlockSpec((1, N, D), lambda b, k, se, sw, th, ch: (b, 0, 0)),
        scratch_shapes=[
            pltpu.VMEM((N, D), jnp.bfloat16),
            pltpu.VMEM((N, D), jnp.bfloat16),
        ],
    )
    return pl.pallas_call(
        _dispatch_body,
        grid_spec=grid_spec,
        out_shape=jax.ShapeDtypeStruct((B, N, D), jnp.float32),
        compiler_params=pltpu.CompilerParams(
            dimension_semantics=("arbitrary", "arbitrary")),
    )(sexp, swts, th, ch, x, tw1, tw2, cw1, cw2)


def kernel(x, router_w, tm_fc1_w, tm_fc1_b, tm_fc2_w, tm_fc2_b,
           cm_fc1_w, cm_fc1_b, cm_fc2_w, cm_fc2_b):
    sexp, swts, aux = _router_call(x, router_w)
    out = _dispatch_call(x, sexp, swts,
                         tm_fc1_w.astype(jnp.bfloat16),
                         tm_fc2_w.astype(jnp.bfloat16),
                         cm_fc1_w.astype(jnp.bfloat16),
                         cm_fc2_w.astype(jnp.bfloat16))
    return out, aux[0, 0]
